# Initial kernel scaffold; baseline (speedup 1.0000x reference)
import jax
import jax.numpy as jnp
from jax.experimental import pallas as pl


def kernel(x, edge_index, node_id, W_gcn, b_gcn, W_mu, b_mu, W_var, b_var, W1, b1, W2, b2):
    raise NotImplementedError("write your pallas kernel here")


# speedup vs baseline: 15.9863x; 15.9863x over previous
import functools

import jax
import jax.numpy as jnp
from jax import lax
from jax.experimental import pallas as pl
from jax.experimental.pallas import tpu as pltpu
from jax.experimental.pallas import tpu_sc as plsc

_N = 10000
_E = 320000
_D = 128
_H = 20
_DEC = 64

_NC = 2
_NS = 16
_NW = _NC * _NS
_EPW = _E // _NW

_CH = 10000
_CL = 2000
_EO = _E // 8
_EP = 327680

_mesh = lambda: plsc.VectorSubcoreMesh(core_axis_name="c", subcore_axis_name="s")
_sc_params = pltpu.CompilerParams(needs_layout_passes=False)


def _zero_vmem(ref, n):
    def zloop(i, carry):
        ref[pl.ds(i * 16, 16)] = jnp.zeros((16,), jnp.float32)
        return carry
    lax.fori_loop(0, n // 16, zloop, 0)


def _sc_deg(dst):
    @functools.partial(
        pl.kernel,
        out_type=jax.ShapeDtypeStruct((_NW, _N), jnp.float32),
        mesh=_mesh(),
        compiler_params=_sc_params,
        scratch_types=[pltpu.VMEM((_EPW,), jnp.int32),
                       pltpu.VMEM((_N,), jnp.float32)],
    )
    def body(dst_hbm, out_hbm, idxb, acc):
        c = lax.axis_index("c")
        s = lax.axis_index("s")
        wid = c * _NS + s
        _zero_vmem(acc, _N)
        pltpu.sync_copy(dst_hbm.at[pl.ds(wid * _EPW, _EPW)], idxb)
        ones = jnp.ones((16,), jnp.float32)

        @plsc.parallel_loop(0, _EPW // 16, unroll=4)
        def iloop(i):
            idx = idxb[pl.ds(i * 16, 16)]
            plsc.addupdate_scatter(acc, [idx], ones)
        pltpu.sync_copy(acc, out_hbm.at[wid])

    return body(dst)


def _sc_agg(src, dst, sxwT):
    _EOA = _E // 8

    @functools.partial(
        pl.kernel,
        out_type=jax.ShapeDtypeStruct((160, _N), jnp.float32),
        mesh=_mesh(),
        compiler_params=_sc_params,
        scratch_types=[pltpu.VMEM((_N,), jnp.float32),
                       pltpu.VMEM((_N,), jnp.float32),
                       pltpu.VMEM((_CH,), jnp.int32),
                       pltpu.VMEM((_CH,), jnp.int32)],
    )
    def body(src_hbm, dst_hbm, sxwT_hbm, out_hbm, colbuf, acc, sbuf, dbuf):
        c = lax.axis_index("c")
        s = lax.axis_index("s")
        wid = c * _NS + s
        for j in range(5):
            u = 5 * wid + j
            col = u // 8
            oct_ = u % 8
            pltpu.sync_copy(sxwT_hbm.at[col], colbuf)
            _zero_vmem(acc, _N)
            def chunk(k, carry):
                e0 = oct_ * _EOA + k * _CH
                pltpu.sync_copy(src_hbm.at[pl.ds(e0, _CH)], sbuf)
                pltpu.sync_copy(dst_hbm.at[pl.ds(e0, _CH)], dbuf)

                @plsc.parallel_loop(0, _CH // 16, unroll=8)
                def iloop(i):
                    si = sbuf[pl.ds(i * 16, 16)]
                    di = dbuf[pl.ds(i * 16, 16)]
                    v = plsc.load_gather(colbuf, [si])
                    plsc.addupdate_scatter(acc, [di], v)
                return carry
            lax.fori_loop(0, _EOA // _CH, chunk, 0)
            pltpu.sync_copy(acc, out_hbm.at[u])

    return body(src, dst, sxwT)


def _sc_edge(src, dst, TsT, TdT):
    @functools.partial(
        pl.kernel,
        out_type=(jax.ShapeDtypeStruct((_H * _EP,), jnp.float32),
                  jax.ShapeDtypeStruct((_H * _EP,), jnp.float32)),
        mesh=_mesh(),
        compiler_params=_sc_params,
        scratch_types=[pltpu.VMEM((5 * _N,), jnp.float32),
                       pltpu.VMEM((5 * _N,), jnp.float32),
                       pltpu.VMEM((_CL,), jnp.int32),
                       pltpu.VMEM((_CL,), jnp.int32),
                       pltpu.VMEM((5 * _CL,), jnp.float32)],
    )
    def body(src_hbm, dst_hbm, TsT_hbm, TdT_hbm, mu_hbm, lv_hbm, scol, dcol, sbuf,
             dbuf, outb):
        c = lax.axis_index("c")
        s = lax.axis_index("s")
        wid = c * _NS + s
        g = wid % 8
        ob = wid // 8
        for j in range(5):
            pltpu.sync_copy(TsT_hbm.at[pl.ds((5 * g + j) * _N, _N)],
                            scol.at[pl.ds(j * _N, _N)])
            pltpu.sync_copy(TdT_hbm.at[pl.ds((5 * g + j) * _N, _N)],
                            dcol.at[pl.ds(j * _N, _N)])
        for oct_i in range(2):
            o = ob + 4 * oct_i
            def chunk(k, carry):
                e0 = o * _EO + k * _CL
                pltpu.sync_copy(src_hbm.at[pl.ds(e0, _CL)], sbuf)
                pltpu.sync_copy(dst_hbm.at[pl.ds(e0, _CL)], dbuf)
                @plsc.parallel_loop(0, _CL // 16, unroll=4)
                def iloop(i):
                    si = sbuf[pl.ds(i * 16, 16)]
                    di = dbuf[pl.ds(i * 16, 16)]
                    for j in range(5):
                        v = (plsc.load_gather(scol, [si + j * _N]) +
                             plsc.load_gather(dcol, [di + j * _N]))
                        outb[pl.ds(j * _CL + i * 16, 16)] = v

                @pl.when(g < 4)
                def _():
                    for j in range(5):
                        pltpu.sync_copy(
                            outb.at[pl.ds(j * _CL, _CL)],
                            mu_hbm.at[pl.ds((5 * g + j) * _EP + e0, _CL)])

                @pl.when(g >= 4)
                def _():
                    for j in range(5):
                        pltpu.sync_copy(
                            outb.at[pl.ds(j * _CL, _CL)],
                            lv_hbm.at[pl.ds((5 * g + j - _H) * _EP + e0, _CL)])
                return carry
            lax.fori_loop(0, _EO // _CL, chunk, 0)

    return body(src, dst, TsT, TdT)


_BN = 2000
_BE = 16384


def _tc_prep(x, W_gcn, degpT):
    def body(degpT_b, x_b, W_b, sxw_b, dis_b):
        deg = 1.0 + jnp.sum(degpT_b[...], axis=1, keepdims=True)
        dis = lax.rsqrt(deg)
        xw = jnp.dot(x_b[...], W_b[...], preferred_element_type=jnp.float32)
        sxw_b[...] = xw * dis
        dis_b[...] = dis

    return pl.pallas_call(
        body,
        grid=(_N // _BN,),
        in_specs=[
            pl.BlockSpec((_BN, _NW), lambda i: (i, 0)),
            pl.BlockSpec((_BN, _D), lambda i: (i, 0)),
            pl.BlockSpec((_D, _H), lambda i: (0, 0)),
        ],
        out_specs=[
            pl.BlockSpec((_BN, _H), lambda i: (i, 0)),
            pl.BlockSpec((_BN, 1), lambda i: (i, 0)),
        ],
        out_shape=[
            jax.ShapeDtypeStruct((_N, _H), jnp.float32),
            jax.ShapeDtypeStruct((_N, 1), jnp.float32),
        ],
    )(degpT, x, W_gcn)


def _tc_tables(aggpT, R, sxw, dis, bg, Wms, Wvs, Wmd, Wvd):
    def body(aggpT_b, R_b, sxw_b, dis_b, bg_b, Wms_b, Wvs_b, Wmd_b, Wvd_b,
             Ts_b, Td_b, x1_b):
        dot = lambda a, b: jnp.dot(a, b, preferred_element_type=jnp.float32)
        agg = dot(aggpT_b[...], R_b[...])
        x1 = jnp.maximum(dis_b[...] * (agg + sxw_b[...]) + bg_b[...], 0.0)
        Ts_b[...] = jnp.concatenate([dot(x1, Wms_b[...]),
                                     dot(x1, Wvs_b[...])], axis=1)
        Td_b[...] = jnp.concatenate([dot(x1, Wmd_b[...]),
                                     dot(x1, Wvd_b[...])], axis=1)
        x1_b[...] = x1

    wspec = pl.BlockSpec((_H, _H), lambda i: (0, 0))
    return pl.pallas_call(
        body,
        grid=(_N // _BN,),
        in_specs=[
            pl.BlockSpec((_BN, 160), lambda i: (i, 0)),
            pl.BlockSpec((160, _H), lambda i: (0, 0)),
            pl.BlockSpec((_BN, _H), lambda i: (i, 0)),
            pl.BlockSpec((_BN, 1), lambda i: (i, 0)),
            pl.BlockSpec((1, _H), lambda i: (0, 0)),
            wspec, wspec, wspec, wspec,
        ],
        out_specs=[
            pl.BlockSpec((_BN, 2 * _H), lambda i: (i, 0)),
            pl.BlockSpec((_BN, 2 * _H), lambda i: (i, 0)),
            pl.BlockSpec((_BN, _H), lambda i: (i, 0)),
        ],
        out_shape=[
            jax.ShapeDtypeStruct((_N, 2 * _H), jnp.float32),
            jax.ShapeDtypeStruct((_N, 2 * _H), jnp.float32),
            jax.ShapeDtypeStruct((_N, _H), jnp.float32),
        ],
    )(aggpT, R, sxw, dis, bg, Wms, Wvs, Wmd, Wvd)


def _tc_edge(muf, lvf, noisef, gate, x1rowT, WmnT, WvnT, bmuT, bvarT,
             W1T, b1T, W2T, b2T):
    nb = _EP // _BE
    rb = _BE // 128

    def body(*refs):
        mus = refs[0:_H]
        lvs = refs[_H:2 * _H]
        noises = refs[2 * _H:3 * _H]
        (gate_b, x1rowT_b, WmnT_b, WvnT_b, bmuT_b, bvarT_b,
         W1T_b, b1T_b, W2T_b, b2T_b, out_b) = refs[3 * _H:]
        dot = lambda a, b: jnp.dot(a, b, preferred_element_type=jnp.float32)
        cmu = dot(WmnT_b[...], x1rowT_b[...]) + bmuT_b[...]
        cvar = dot(WvnT_b[...], x1rowT_b[...]) + bvarT_b[...]
        zs = []
        for r in range(_H):
            mu_r = mus[r][...] + cmu[r:r + 1, :]
            lv_r = lvs[r][...] + cvar[r:r + 1, :]
            z_r = mu_r + jnp.exp(lv_r) * noises[r][...]
            zs.append(z_r.reshape(1, _BE))
        z = jnp.concatenate(zs, axis=0)
        h = jnp.maximum(dot(W1T_b[...], z) + b1T_b[...], 0.0)
        o = jnp.maximum(dot(W2T_b[...], h) + b2T_b[...], 0.0)
        out_b[...] = jax.nn.sigmoid(gate_b[...] + o)

    row_specs = lambda: [
        pl.BlockSpec((rb, 128), (lambda i, r=r: (r * nb + i, 0)))
        for r in range(_H)]
    return pl.pallas_call(
        body,
        grid=(nb,),
        in_specs=row_specs() + row_specs() + row_specs() + [
            pl.BlockSpec((1, _BE), lambda i: (0, i)),
            pl.BlockSpec((_H, 1), lambda i: (0, 0)),
            pl.BlockSpec((_H, _H), lambda i: (0, 0)),
            pl.BlockSpec((_H, _H), lambda i: (0, 0)),
            pl.BlockSpec((_H, 1), lambda i: (0, 0)),
            pl.BlockSpec((_H, 1), lambda i: (0, 0)),
            pl.BlockSpec((_DEC, _H), lambda i: (0, 0)),
            pl.BlockSpec((_DEC, 1), lambda i: (0, 0)),
            pl.BlockSpec((1, _DEC), lambda i: (0, 0)),
            pl.BlockSpec((1, 1), lambda i: (0, 0)),
        ],
        out_specs=pl.BlockSpec((1, _BE), lambda i: (0, i)),
        out_shape=jax.ShapeDtypeStruct((1, _EP), jnp.float32),
    )(*([muf] * _H), *([lvf] * _H), *([noisef] * _H), gate, x1rowT, WmnT,
      WvnT, bmuT, bvarT, W1T, b1T, W2T, b2T)


def kernel(x, edge_index, node_id, W_gcn, b_gcn, W_mu, b_mu, W_var, b_var,
           W1, b1, W2, b2):
    src = edge_index[0]
    dst = edge_index[1]

    degp = _sc_deg(dst)
    sxw, dis = _tc_prep(x, W_gcn, degp.T)
    aggp = _sc_agg(src, dst, sxw.T)
    R = (jnp.arange(160)[:, None] // 8 ==
         jnp.arange(_H)[None, :]).astype(jnp.float32)
    Ts, Td, x1 = _tc_tables(
        aggp.T, R, sxw, dis, b_gcn.reshape(1, _H),
        W_mu[:_H], W_var[:_H], W_mu[_H:2 * _H], W_var[_H:2 * _H])
    x1row = lax.dynamic_slice(x1, (node_id, 0), (1, _H))
    muf, lvf = _sc_edge(src, dst, Ts.T.reshape(-1), Td.T.reshape(-1))
    mu2 = muf.reshape(_H * _EP // 128, 128)
    lv2 = lvf.reshape(_H * _EP // 128, 128)

    noiseT = jax.random.normal(
        jax.random.key(42), (_E * _H,), jnp.float32).reshape(_E, _H).T
    noise2 = jnp.pad(noiseT, ((0, 0), (0, _EP - _E))).reshape(
        _H * _EP // 128, 128)
    u = jax.random.uniform(jax.random.key(7), (1, _E), jnp.float32)
    bias = 0.0 + 0.0001
    eps = (bias - (1.0 - bias)) * u + (1.0 - bias)
    gate = jnp.pad(jnp.log(eps) - jnp.log(1.0 - eps),
                   ((0, 0), (0, _EP - _E)))

    graphT = _tc_edge(mu2, lv2, noise2, gate, x1row.reshape(_H, 1),
                      W_mu[2 * _H:].T, W_var[2 * _H:].T,
                      b_mu.reshape(_H, 1), b_var.reshape(_H, 1),
                      W1.T, b1.reshape(_DEC, 1), W2.T, b2.reshape(1, 1))
    return graphT[:, :_E].reshape(_E, 1)
